# baseline (speedup 1.0000x reference)
import functools

import jax
import jax.numpy as jnp
from jax import lax
from jax.experimental import pallas as pl
from jax.experimental.pallas import tpu as pltpu
from jax.experimental.pallas import tpu_sc as plsc

NC = 2
NS = 16
LANES = 16
CHUNK = 128
IB = 8


def _sc_segsum_body(n_pad, ch0, ch1, d, compute_deg, x_hbm, src_hbm,
                    dst_hbm, *refs):
    if compute_deg:
        (part_out, deg_out, idx_src_v, idx_dst_v, rows_v, acc_sh, deg_v,
         sem, ssem) = refs
    else:
        part_out, idx_src_v, idx_dst_v, rows_v, acc_sh, sem, ssem = refs
        deg_out = deg_v = None

    core = lax.axis_index("c")
    sub = lax.axis_index("s")
    wid = core * NS + sub
    rpt = n_pad // NS

    zeros16 = jnp.zeros((LANES,), jnp.float32)
    ones16 = jnp.ones((LANES,), jnp.float32)

    def _zrows(i, carry):
        for j in range(d // LANES):
            rows_v[0, i, pl.ds(j * LANES, LANES)] = zeros16
        return carry
    lax.fori_loop(0, CHUNK, _zrows, 0)

    base = sub * rpt
    for k in range(rpt // CHUNK):
        pltpu.sync_copy(rows_v.at[0],
                        acc_sh.at[pl.ds(base + k * CHUNK, CHUNK)])

    if compute_deg:
        def _zdeg(i, carry):
            deg_v[pl.ds(i * LANES, LANES)] = zeros16
            return carry
        lax.fori_loop(0, n_pad // LANES, _zdeg, 0)

    plsc.subcore_barrier()

    ch = jnp.where(core == 0, ch0, ch1)
    nblk = ch // IB

    def _stage(blk, slot):
        pltpu.sync_copy(src_hbm.at[wid, pl.ds(blk * IB, IB)],
                        idx_src_v.at[slot])
        pltpu.sync_copy(dst_hbm.at[wid, pl.ds(blk * IB, IB)],
                        idx_dst_v.at[slot])

    _stage(0, 0)
    pltpu.async_copy(x_hbm.at[idx_src_v.at[0, 0]], rows_v.at[0], sem.at[0])

    def _chunk(c, carry):
        blk = c // IB
        j = c - blk * IB
        par = lax.rem(blk, 2)
        buf = lax.rem(c, 2)

        @pl.when(jnp.logical_and(j == 0, blk + 1 < nblk))
        def _():
            _stage(blk + 1, lax.rem(blk + 1, 2))

        pltpu.make_async_copy(x_hbm.at[idx_src_v.at[par, j]],
                              rows_v.at[buf], sem.at[buf]).wait()

        c1 = c + 1
        blk1 = c1 // IB
        j1 = c1 - blk1 * IB
        par1 = lax.rem(blk1, 2)
        buf1 = lax.rem(c1, 2)

        @pl.when(c1 < ch)
        def _():
            pltpu.async_copy(x_hbm.at[idx_src_v.at[par1, j1]],
                             rows_v.at[buf1], sem.at[buf1])

        desc = pltpu.async_copy(rows_v.at[buf],
                                acc_sh.at[idx_dst_v.at[par, j]],
                                ssem, add=True)
        if compute_deg:
            for q in range(CHUNK // LANES):
                idx16 = idx_dst_v[par, j, pl.ds(q * LANES, LANES)]
                plsc.addupdate_scatter(deg_v, [idx16], ones16)
        desc.wait()
        return carry
    lax.fori_loop(0, ch, _chunk, 0)

    plsc.subcore_barrier()

    for k in range(rpt // CHUNK):
        sl = pl.ds(base + k * CHUNK, CHUNK)
        pltpu.sync_copy(acc_sh.at[sl], rows_v.at[k % 2])
        pltpu.sync_copy(rows_v.at[k % 2], part_out.at[core, sl])
    if compute_deg:
        pltpu.sync_copy(deg_v, deg_out.at[wid])


def _make_sc_segsum(n_pad, ch0, ch1, d, compute_deg=False):
    mesh = plsc.VectorSubcoreMesh(core_axis_name="c", subcore_axis_name="s",
                                  num_cores=NC, num_subcores=NS)
    ch_max = max(ch0, ch1)
    part_t = jax.ShapeDtypeStruct((NC, n_pad, d), jnp.float32)
    scratch = [
        pltpu.VMEM((2, IB, CHUNK), jnp.int32),
        pltpu.VMEM((2, IB, CHUNK), jnp.int32),
        pltpu.VMEM((2, CHUNK, d), jnp.float32),
        pltpu.VMEM_SHARED((n_pad, d), jnp.float32),
    ]
    if compute_deg:
        out_type = (part_t,
                    jax.ShapeDtypeStruct((NC * NS, n_pad), jnp.float32))
        scratch.append(pltpu.VMEM((n_pad,), jnp.float32))
    else:
        out_type = part_t
    scratch.append(pltpu.SemaphoreType.DMA((2,)))
    scratch.append(pltpu.SemaphoreType.DMA)
    del ch_max
    return pl.kernel(
        functools.partial(_sc_segsum_body, n_pad, ch0, ch1, d, compute_deg),
        out_type=out_type, mesh=mesh, scratch_types=scratch,
        compiler_params=pltpu.CompilerParams(needs_layout_passes=False))


def _tc_layer_body(relu, p_ref, degp_ref, h_ref, wl_ref, b_ref, wr_ref,
                   o_ref):
    deg = jnp.sum(degp_ref[...], axis=0)
    s = p_ref[0] + p_ref[1]
    agg = s / jnp.maximum(deg, 1.0)[:, None]
    acc = lax.dot_general(agg, wl_ref[...], (((1,), (1,)), ((), ())),
                          preferred_element_type=jnp.float32)
    acc = acc + lax.dot_general(h_ref[...], wr_ref[...],
                                (((1,), (1,)), ((), ())),
                                preferred_element_type=jnp.float32)
    acc = acc + b_ref[...]
    o_ref[...] = jnp.maximum(acc, 0.0) if relu else acc


def _tc_layer(p, degp, h, w_l, b, w_r, relu, bn):
    n, d = h.shape
    o = w_l.shape[0]
    return pl.pallas_call(
        functools.partial(_tc_layer_body, relu),
        grid=(n // bn,),
        in_specs=[
            pl.BlockSpec((NC, bn, d), lambda i: (0, i, 0)),
            pl.BlockSpec((NC * NS, bn), lambda i: (0, i)),
            pl.BlockSpec((bn, d), lambda i: (i, 0)),
            pl.BlockSpec((o, d), lambda i: (0, 0)),
            pl.BlockSpec((1, o), lambda i: (0, 0)),
            pl.BlockSpec((o, d), lambda i: (0, 0)),
        ],
        out_specs=pl.BlockSpec((bn, o), lambda i: (i, 0)),
        out_shape=jax.ShapeDtypeStruct((n, o), jnp.float32),
    )(p, degp, h, w_l, b.reshape(1, o), w_r)


CORE0_FRAC = 0.5


def _edge_layout(idx, e, cap, ch0, ch1, fill):
    flat = jnp.concatenate(
        [idx, jnp.full((cap * CHUNK - e,), fill, jnp.int32)]
    ).reshape(cap, CHUNK)
    ch_max = max(ch0, ch1)
    a = jnp.pad(flat[:NS * ch0].reshape(NS, ch0, CHUNK),
                ((0, 0), (0, ch_max - ch0), (0, 0)))
    b = jnp.pad(flat[NS * ch0:].reshape(NS, ch1, CHUNK),
                ((0, 0), (0, ch_max - ch1), (0, 0)))
    return jnp.concatenate([a, b], axis=0)


def kernel(x, edge_index, W_l1, b1, W_r1, W_l2, b2, W_r2):
    n, d = x.shape
    e = edge_index.shape[1]

    rpt = -(-n // NS)
    rpt = -(-rpt // CHUNK) * CHUNK
    n_pad = rpt * NS
    tc_total = -(-e // CHUNK)
    ch0 = max(IB, int(round(tc_total * CORE0_FRAC / NS / IB)) * IB)
    rem = max(0, tc_total - NS * ch0)
    per_tile1 = -(-rem // NS)
    ch1 = max(IB, -(-per_tile1 // IB) * IB)
    cap = NS * (ch0 + ch1)

    src = edge_index[0].astype(jnp.int32)
    dst = edge_index[1].astype(jnp.int32)
    src_r = _edge_layout(src, e, cap, ch0, ch1, 0)
    dst_r = _edge_layout(dst, e, cap, ch0, ch1, n)

    x_p = jnp.pad(x, ((0, n_pad - n), (0, 0)))

    bn = n_pad // 5 if (n_pad // 5) % 128 == 0 else n_pad

    p1, degp = _make_sc_segsum(n_pad, ch0, ch1, d, True)(x_p, src_r, dst_r)
    h = _tc_layer(p1, degp, x_p, W_l1, b1, W_r1, True, bn)
    p2 = _make_sc_segsum(n_pad, ch0, ch1, d)(h, src_r, dst_r)
    out = _tc_layer(p2, degp, h, W_l2, b2, W_r2, False, bn)
    return out[:n]


# speedup vs baseline: 1.0465x; 1.0465x over previous
import functools

import jax
import jax.numpy as jnp
from jax import lax
from jax.experimental import pallas as pl
from jax.experimental.pallas import tpu as pltpu
from jax.experimental.pallas import tpu_sc as plsc

NC = 2
NS = 16
LANES = 16
CHUNK = 128
IB = 8


def _sc_segsum_body(n_pad, ch0, ch1, d, compute_deg, x_hbm, src_hbm,
                    dst_hbm, *refs):
    if compute_deg:
        (part_out, deg_out, idx_src_v, idx_dst_v, rows_v, acc_sh, deg_v,
         sem) = refs
    else:
        part_out, idx_src_v, idx_dst_v, rows_v, acc_sh, sem = refs
        deg_out = deg_v = None

    core = lax.axis_index("c")
    sub = lax.axis_index("s")
    wid = core * NS + sub
    rpt = n_pad // NS

    zeros16 = jnp.zeros((LANES,), jnp.float32)
    ones16 = jnp.ones((LANES,), jnp.float32)

    def _zrows(i, carry):
        for j in range(d // LANES):
            rows_v[0, i, pl.ds(j * LANES, LANES)] = zeros16
        return carry
    lax.fori_loop(0, CHUNK, _zrows, 0)

    base = sub * rpt
    for k in range(rpt // CHUNK):
        pltpu.sync_copy(rows_v.at[0],
                        acc_sh.at[pl.ds(base + k * CHUNK, CHUNK)])

    if compute_deg:
        def _zdeg(i, carry):
            deg_v[pl.ds(i * LANES, LANES)] = zeros16
            return carry
        lax.fori_loop(0, n_pad // LANES, _zdeg, 0)

    plsc.subcore_barrier()

    ch = jnp.where(core == 0, ch0, ch1)
    nblk = ch // IB

    def _stage(blk, slot):
        pltpu.sync_copy(src_hbm.at[wid, pl.ds(blk * IB, IB)],
                        idx_src_v.at[slot])
        pltpu.sync_copy(dst_hbm.at[wid, pl.ds(blk * IB, IB)],
                        idx_dst_v.at[slot])

    _stage(0, 0)
    pltpu.async_copy(x_hbm.at[idx_src_v.at[0, 0]], rows_v.at[0], sem.at[0])

    def _chunk(c, carry):
        blk = c // IB
        j = c - blk * IB
        par = lax.rem(blk, 2)
        buf = lax.rem(c, 2)

        @pl.when(jnp.logical_and(j == 0, blk + 1 < nblk))
        def _():
            _stage(blk + 1, lax.rem(blk + 1, 2))

        c1 = c + 1
        blk1 = c1 // IB
        j1 = c1 - blk1 * IB
        par1 = lax.rem(blk1, 2)
        buf1 = lax.rem(c1, 2)

        @pl.when(c1 < ch)
        def _():
            pltpu.async_copy(x_hbm.at[idx_src_v.at[par1, j1]],
                             rows_v.at[buf1], sem.at[buf1])

        pltpu.make_async_copy(x_hbm.at[idx_src_v.at[par, j]],
                              rows_v.at[buf], sem.at[buf]).wait()

        pltpu.sync_copy(rows_v.at[buf], acc_sh.at[idx_dst_v.at[par, j]],
                        add=True)
        if compute_deg:
            for q in range(CHUNK // LANES):
                idx16 = idx_dst_v[par, j, pl.ds(q * LANES, LANES)]
                plsc.addupdate_scatter(deg_v, [idx16], ones16)
        return carry
    lax.fori_loop(0, ch, _chunk, 0)

    plsc.subcore_barrier()

    for k in range(rpt // CHUNK):
        sl = pl.ds(base + k * CHUNK, CHUNK)
        pltpu.sync_copy(acc_sh.at[sl], rows_v.at[k % 2])
        pltpu.sync_copy(rows_v.at[k % 2], part_out.at[core, sl])
    if compute_deg:
        pltpu.sync_copy(deg_v, deg_out.at[wid])


def _make_sc_segsum(n_pad, ch0, ch1, d, compute_deg=False):
    mesh = plsc.VectorSubcoreMesh(core_axis_name="c", subcore_axis_name="s",
                                  num_cores=NC, num_subcores=NS)
    ch_max = max(ch0, ch1)
    part_t = jax.ShapeDtypeStruct((NC, n_pad, d), jnp.float32)
    scratch = [
        pltpu.VMEM((2, IB, CHUNK), jnp.int32),
        pltpu.VMEM((2, IB, CHUNK), jnp.int32),
        pltpu.VMEM((2, CHUNK, d), jnp.float32),
        pltpu.VMEM_SHARED((n_pad, d), jnp.float32),
    ]
    if compute_deg:
        out_type = (part_t,
                    jax.ShapeDtypeStruct((NC * NS, n_pad), jnp.float32))
        scratch.append(pltpu.VMEM((n_pad,), jnp.float32))
    else:
        out_type = part_t
    scratch.append(pltpu.SemaphoreType.DMA((2,)))
    del ch_max
    return pl.kernel(
        functools.partial(_sc_segsum_body, n_pad, ch0, ch1, d, compute_deg),
        out_type=out_type, mesh=mesh, scratch_types=scratch,
        compiler_params=pltpu.CompilerParams(needs_layout_passes=False))


def _tc_layer_body(relu, p_ref, degp_ref, h_ref, wl_ref, b_ref, wr_ref,
                   o_ref):
    deg = jnp.sum(degp_ref[...], axis=0)
    s = p_ref[0] + p_ref[1]
    agg = s / jnp.maximum(deg, 1.0)[:, None]
    acc = lax.dot_general(agg, wl_ref[...], (((1,), (1,)), ((), ())),
                          preferred_element_type=jnp.float32)
    acc = acc + lax.dot_general(h_ref[...], wr_ref[...],
                                (((1,), (1,)), ((), ())),
                                preferred_element_type=jnp.float32)
    acc = acc + b_ref[...]
    o_ref[...] = jnp.maximum(acc, 0.0) if relu else acc


def _tc_layer(p, degp, h, w_l, b, w_r, relu, bn):
    n, d = h.shape
    o = w_l.shape[0]
    return pl.pallas_call(
        functools.partial(_tc_layer_body, relu),
        grid=(n // bn,),
        in_specs=[
            pl.BlockSpec((NC, bn, d), lambda i: (0, i, 0)),
            pl.BlockSpec((NC * NS, bn), lambda i: (0, i)),
            pl.BlockSpec((bn, d), lambda i: (i, 0)),
            pl.BlockSpec((o, d), lambda i: (0, 0)),
            pl.BlockSpec((1, o), lambda i: (0, 0)),
            pl.BlockSpec((o, d), lambda i: (0, 0)),
        ],
        out_specs=pl.BlockSpec((bn, o), lambda i: (i, 0)),
        out_shape=jax.ShapeDtypeStruct((n, o), jnp.float32),
    )(p, degp, h, w_l, b.reshape(1, o), w_r)


CORE0_FRAC = 0.5


def _edge_layout(idx, e, cap, ch0, ch1, fill):
    flat = jnp.concatenate(
        [idx, jnp.full((cap * CHUNK - e,), fill, jnp.int32)]
    ).reshape(cap, CHUNK)
    ch_max = max(ch0, ch1)
    a = jnp.pad(flat[:NS * ch0].reshape(NS, ch0, CHUNK),
                ((0, 0), (0, ch_max - ch0), (0, 0)))
    b = jnp.pad(flat[NS * ch0:].reshape(NS, ch1, CHUNK),
                ((0, 0), (0, ch_max - ch1), (0, 0)))
    return jnp.concatenate([a, b], axis=0)


def kernel(x, edge_index, W_l1, b1, W_r1, W_l2, b2, W_r2):
    n, d = x.shape
    e = edge_index.shape[1]

    rpt = -(-n // NS)
    rpt = -(-rpt // CHUNK) * CHUNK
    n_pad = rpt * NS
    tc_total = -(-e // CHUNK)
    ch0 = max(IB, int(round(tc_total * CORE0_FRAC / NS / IB)) * IB)
    rem = max(0, tc_total - NS * ch0)
    per_tile1 = -(-rem // NS)
    ch1 = max(IB, -(-per_tile1 // IB) * IB)
    cap = NS * (ch0 + ch1)

    src = edge_index[0].astype(jnp.int32)
    dst = edge_index[1].astype(jnp.int32)
    src_r = _edge_layout(src, e, cap, ch0, ch1, 0)
    dst_r = _edge_layout(dst, e, cap, ch0, ch1, n)

    x_p = jnp.pad(x, ((0, n_pad - n), (0, 0)))

    bn = n_pad // 5 if (n_pad // 5) % 128 == 0 else n_pad

    p1, degp = _make_sc_segsum(n_pad, ch0, ch1, d, True)(x_p, src_r, dst_r)
    h = _tc_layer(p1, degp, x_p, W_l1, b1, W_r1, True, bn)
    p2 = _make_sc_segsum(n_pad, ch0, ch1, d)(h, src_r, dst_r)
    out = _tc_layer(p2, degp, h, W_l2, b2, W_r2, False, bn)
    return out[:n]
